# baseline (speedup 1.0000x reference)
import functools

import jax
import jax.numpy as jnp
from jax import lax
from jax.experimental import pallas as pl
from jax.experimental.pallas import tpu as pltpu
from jax.experimental.pallas import tpu_sc as plsc

N = 50000
E = 800000
S = 10
C = 64
F = 32
G = 500
R_MAX = 5.0

CE = 2048
CN = 1024
E_PAD = 819200
NROWS = E_PAD // 128
N_PAD = 51200
NNR = N_PAD // 128

NC, NS = 2, 16
NW = NC * NS
ROWS_PW = NROWS // NW

_MESH = plsc.VectorSubcoreMesh(core_axis_name="c", subcore_axis_name="s",
                               num_cores=NC, num_subcores=NS)
_SC_PARAMS = pltpu.CompilerParams(needs_layout_passes=False,
                                  use_tc_tiling_on_sc=False)


def _kn0_body(na_ref, ae_ref, sp_ref, eat_ref):
    na = na_ref[...]
    iota = lax.broadcasted_iota(jnp.int32, (S, 1), 0).astype(jnp.float32)
    spf = jnp.dot(na, iota)
    sp_ref[...] = (spf + 0.5).astype(jnp.int32).reshape(CN // 128, 128)
    eat_ref[...] = jnp.dot(na, ae_ref[...]).reshape(CN // 128, 128)


def _edge_feats(evx_ref, evy_ref, evz_ref, el_ref):
    r = el_ref[...]
    rs = jnp.maximum(r, 1e-6)
    x = r * (1.0 / R_MAX)
    x2 = x * x
    x6 = x2 * x2 * x2
    env = 1.0 - 28.0 * x6 + 48.0 * x6 * x - 21.0 * x6 * x2
    env = jnp.where(x < 1.0, env, 0.0)
    ki = lax.broadcasted_iota(jnp.int32, (1, F), 1)
    nrow = (ki // 4 + 1).astype(jnp.float32)
    lrow = ki % 4
    arg = (jnp.pi / R_MAX) * (rs * nrow)
    rb = (jnp.sqrt(2.0 / R_MAX) * jnp.sin(arg)) / rs
    ux = evx_ref[...] / rs
    uy = evy_ref[...] / rs
    uz = evz_ref[...] / rs
    sh = jnp.where(lrow == 0, 1.0,
                   jnp.where(lrow == 1, ux, jnp.where(lrow == 2, uy, uz)))
    return rb * env * sh


def _ke1_body(evx_ref, evy_ref, evz_ref, el_ref, sp_ref, wr0_ref, wn_ref,
              m3_ref):
    ef = _edge_feats(evx_ref, evy_ref, evz_ref, el_ref)
    rw_t = lax.dot_general(wr0_ref[...], ef, (((0,), (1,)), ((), ())))
    onehot = (sp_ref[...] == lax.broadcasted_iota(jnp.int32, (CE, S), 1)
              ).astype(jnp.float32)
    wn_t = lax.dot_general(wn_ref[...], onehot, (((0,), (1,)), ((), ())))
    m3_ref[...] = (rw_t * wn_t).reshape(C, CE // 128, 128)


def _kn2_body(agt_ref, sp_ref, wo0_ref, sk0_ref, wu_ref, sr1_ref,
              wr0c_ref, u_ref, e0_ref, e1s_ref):
    onehot = (sp_ref[...] == lax.broadcasted_iota(jnp.int32, (CN, S), 1)
              ).astype(jnp.float32)
    nf1 = (lax.dot_general(agt_ref[...], wo0_ref[...], (((0,), (0,)), ((), ())))
           + jnp.dot(onehot, sk0_ref[...]))
    e0_ref[...] = jnp.dot(nf1, wr0c_ref[...]).reshape(CN // 128, 128)
    u_ref[...] = jnp.dot(nf1, wu_ref[...])
    e1s = jnp.sum(nf1 * jnp.dot(onehot, sr1_ref[...]), axis=1, keepdims=True)
    e1s_ref[...] = e1s.reshape(CN // 128, 128)


def _ke2_body(evx_ref, evy_ref, evz_ref, el_ref, ur_ref, es_ref):
    ef = _edge_feats(evx_ref, evy_ref, evz_ref, el_ref)
    es = jnp.sum(ef * ur_ref[...], axis=1, keepdims=True)
    es_ref[...] = es.reshape(CE // 128, 128)


def _kn3_body(eat_ref, e0_ref, e1_ref, batch_ref, acc_ref):
    i = pl.program_id(0)
    onehot = (batch_ref[...] == lax.broadcasted_iota(jnp.int32, (CN, G), 1)
              ).astype(jnp.float32)
    cdims = (((0,), (0,)), ((), ()))
    a0 = lax.dot_general(onehot, eat_ref[...], cdims)
    a1 = lax.dot_general(onehot, e0_ref[...], cdims)
    a2 = lax.dot_general(onehot, e1_ref[...], cdims)
    col = lax.broadcasted_iota(jnp.int32, (1, 3), 1)
    sel0 = (col == 0).astype(jnp.float32)
    sel1 = (col == 1).astype(jnp.float32)
    sel2 = (col == 2).astype(jnp.float32)

    @pl.when(i == 0)
    def _():
        acc_ref[...] = jnp.zeros((G, 3), jnp.float32)

    acc_ref[...] += a0 * sel0 + a1 * sel1 + a2 * sel2


def _ks0_body(src2d, species_hbm, out_hbm, tab_v, idx_v, out_v):
    c = lax.axis_index("c")
    s = lax.axis_index("s")
    wid = s * NC + c
    rbase = wid * ROWS_PW
    pltpu.sync_copy(species_hbm, tab_v)

    def outer(jo, carry):
        pltpu.sync_copy(src2d.at[pl.ds(rbase + jo * 8, 8)], idx_v)
        for r in range(8):
            for g in range(8):
                idx = idx_v[r, pl.ds(g * 16, 16)]
                out_v[r, pl.ds(g * 16, 16)] = plsc.load_gather(tab_v, [idx])
        pltpu.sync_copy(out_v, out_hbm.at[pl.ds(rbase + jo * 8, 8)])
        return carry

    lax.fori_loop(0, ROWS_PW // 8, outer, 0)


_KS1_CH = 32

def _ks1_body(m3_hbm, dst2d, out_hbm, acc0, acc1, idx_v, v0, v1):
    c = lax.axis_index("c")
    s = lax.axis_index("s")
    wid = s * NC + c

    def z(i, carry):
        acc0[pl.ds(i * 16, 16)] = jnp.zeros((16,), jnp.float32)
        acc1[pl.ds(i * 16, 16)] = jnp.zeros((16,), jnp.float32)
        return carry

    lax.fori_loop(0, N_PAD // 16, z, 0)

    def outer(jo, carry):
        row0 = jo * _KS1_CH
        pltpu.sync_copy(dst2d.at[pl.ds(row0, _KS1_CH)], idx_v)
        pltpu.sync_copy(m3_hbm.at[2 * wid].at[pl.ds(row0, _KS1_CH)], v0)
        pltpu.sync_copy(m3_hbm.at[2 * wid + 1].at[pl.ds(row0, _KS1_CH)], v1)

        def row(r, carry2):
            for g in range(8):
                vidx = idx_v[r, pl.ds(g * 16, 16)]
                plsc.addupdate_scatter(acc0, [vidx], v0[r, pl.ds(g * 16, 16)])
                plsc.addupdate_scatter(acc1, [vidx], v1[r, pl.ds(g * 16, 16)])
            return carry2

        lax.fori_loop(0, _KS1_CH, row, 0)
        return carry

    lax.fori_loop(0, NROWS // _KS1_CH, outer, 0)
    pltpu.sync_copy(acc0, out_hbm.at[2 * wid])
    pltpu.sync_copy(acc1, out_hbm.at[2 * wid + 1])


def _ks2_body(src2d, u_hbm, out_hbm, idx_v, rows_v, sem):
    c = lax.axis_index("c")
    s = lax.axis_index("s")
    wid = s * NC + c
    rbase = wid * ROWS_PW

    def outer(jo, carry):
        row0 = rbase + jo * 8
        pltpu.sync_copy(src2d.at[pl.ds(row0, 8)], idx_v)
        for k in range(8):
            pltpu.async_copy(u_hbm.at[idx_v.at[k]],
                             rows_v.at[pl.ds(k * 128, 128)], sem).wait()
        pltpu.sync_copy(rows_v, out_hbm.at[pl.ds(row0 * 128, 1024)])
        return carry

    lax.fori_loop(0, ROWS_PW // 8, outer, 0)


def _ks3_body(dst2d, es2d, out_hbm, d1_v, val_v, idx_v):
    c = lax.axis_index("c")
    s = lax.axis_index("s")
    wid = s * NC + c
    rbase = wid * ROWS_PW

    def z(i, carry):
        d1_v[pl.ds(i * 16, 16)] = jnp.zeros((16,), jnp.float32)
        return carry

    lax.fori_loop(0, N // 16, z, 0)

    def outer(jo, carry):
        row0 = rbase + jo * 8
        pltpu.sync_copy(dst2d.at[pl.ds(row0, 8)], idx_v)
        pltpu.sync_copy(es2d.at[pl.ds(row0, 8)], val_v)
        for k in range(8):
            for g in range(8):
                vidx = idx_v[k, pl.ds(g * 16, 16)]
                vval = val_v[k, pl.ds(g * 16, 16)]
                plsc.addupdate_scatter(d1_v, [vidx], vval)
        return carry

    lax.fori_loop(0, ROWS_PW // 8, outer, 0)
    pltpu.sync_copy(d1_v, out_hbm.at[pl.ds(wid * N, N)])


def kernel(node_attrs, edge_vectors, edge_lengths, edge_index, batch,
           num_graphs, W_node, atomic_energies,
           W_r0, W_skip0, W_out0, w_read0,
           W_r1, W_skip1, W_out1, w_read1):
    f32 = jnp.float32
    skip0_tab = jnp.einsum('ac,acd->ad', W_node, W_skip0)
    v1 = W_out1 @ w_read1
    sr1_tab = jnp.einsum('acd,d->ac', W_skip1, w_read1)
    Wu = (W_r1 * v1[None, :]).T

    src = edge_index[0]
    dst = edge_index[1]
    pad = E_PAD - E
    src_p = jnp.pad(src, (0, pad)).astype(jnp.int32)
    dst_p = jnp.pad(dst, (0, pad)).astype(jnp.int32)
    el_p = jnp.pad(edge_lengths, (0, pad), constant_values=2.0 * R_MAX)
    ev_p = jnp.pad(edge_vectors, ((0, pad), (0, 0)), constant_values=1.0)
    evx = ev_p[:, 0:1]
    evy = ev_p[:, 1:2]
    evz = ev_p[:, 2:3]
    el_col = el_p[:, None]
    src2d = src_p.reshape(NROWS, 128)
    dst2d = dst_p.reshape(NROWS, 128)
    na_pad = jnp.pad(node_attrs, ((0, N_PAD - N), (0, 0)))
    batch_pad = jnp.pad(batch.astype(jnp.int32), (0, N_PAD - N),
                        constant_values=G)[:, None]

    species2, eat2 = pl.pallas_call(
        _kn0_body,
        grid=(N_PAD // CN,),
        in_specs=[pl.BlockSpec((CN, S), lambda i: (i, 0)),
                  pl.BlockSpec((S, 1), lambda i: (0, 0))],
        out_specs=[pl.BlockSpec((CN // 128, 128), lambda i: (i, 0)),
                   pl.BlockSpec((CN // 128, 128), lambda i: (i, 0))],
        out_shape=[jax.ShapeDtypeStruct((NNR, 128), jnp.int32),
                   jax.ShapeDtypeStruct((NNR, 128), f32)],
    )(na_pad, atomic_energies[:, None])
    species_flat = species2.reshape(N_PAD)
    species_col = species2.reshape(N_PAD, 1)
    eat_col = eat2.reshape(N_PAD, 1)

    ks0 = functools.partial(
        pl.kernel,
        out_type=jax.ShapeDtypeStruct((NROWS, 128), jnp.int32),
        mesh=_MESH,
        compiler_params=_SC_PARAMS,
        scratch_types=[pltpu.VMEM((N_PAD,), jnp.int32),
                       pltpu.VMEM((8, 128), jnp.int32),
                       pltpu.VMEM((8, 128), jnp.int32)],
    )(_ks0_body)
    sp2d = ks0(src2d, species_flat)
    sp_col = sp2d.reshape(E_PAD, 1)

    m3 = pl.pallas_call(
        _ke1_body,
        grid=(E_PAD // CE,),
        in_specs=[pl.BlockSpec((CE, 1), lambda i: (i, 0)),
                  pl.BlockSpec((CE, 1), lambda i: (i, 0)),
                  pl.BlockSpec((CE, 1), lambda i: (i, 0)),
                  pl.BlockSpec((CE, 1), lambda i: (i, 0)),
                  pl.BlockSpec((CE, 1), lambda i: (i, 0)),
                  pl.BlockSpec((F, C), lambda i: (0, 0)),
                  pl.BlockSpec((S, C), lambda i: (0, 0))],
        out_specs=pl.BlockSpec((C, CE // 128, 128), lambda i: (0, i, 0)),
        out_shape=jax.ShapeDtypeStruct((C, NROWS, 128), f32),
    )(evx, evy, evz, el_col, sp_col, W_r0, W_node)

    ks1 = functools.partial(
        pl.kernel,
        out_type=jax.ShapeDtypeStruct((C, N_PAD), f32),
        mesh=_MESH,
        compiler_params=_SC_PARAMS,
        scratch_types=[pltpu.VMEM((N_PAD,), f32),
                       pltpu.VMEM((N_PAD,), f32),
                       pltpu.VMEM((_KS1_CH, 128), jnp.int32),
                       pltpu.VMEM((_KS1_CH, 128), f32),
                       pltpu.VMEM((_KS1_CH, 128), f32)],
    )(_ks1_body)
    agg_t = ks1(m3, dst2d)

    u_nodes, e02, e1s2 = pl.pallas_call(
        _kn2_body,
        grid=(N_PAD // CN,),
        in_specs=[pl.BlockSpec((C, CN), lambda i: (0, i)),
                  pl.BlockSpec((CN, 1), lambda i: (i, 0)),
                  pl.BlockSpec((C, C), lambda i: (0, 0)),
                  pl.BlockSpec((S, C), lambda i: (0, 0)),
                  pl.BlockSpec((C, F), lambda i: (0, 0)),
                  pl.BlockSpec((S, C), lambda i: (0, 0)),
                  pl.BlockSpec((C, 1), lambda i: (0, 0))],
        out_specs=[pl.BlockSpec((CN, F), lambda i: (i, 0)),
                   pl.BlockSpec((CN // 128, 128), lambda i: (i, 0)),
                   pl.BlockSpec((CN // 128, 128), lambda i: (i, 0))],
        out_shape=[jax.ShapeDtypeStruct((N_PAD, F), f32),
                   jax.ShapeDtypeStruct((NNR, 128), f32),
                   jax.ShapeDtypeStruct((NNR, 128), f32)],
    )(agg_t, species_col, W_out0, skip0_tab, Wu, sr1_tab, w_read0[:, None])

    ks2 = functools.partial(
        pl.kernel,
        out_type=jax.ShapeDtypeStruct((E_PAD, F), f32),
        mesh=_MESH,
        compiler_params=_SC_PARAMS,
        scratch_types=[pltpu.VMEM((8, 128), jnp.int32),
                       pltpu.VMEM((1024, F), f32),
                       pltpu.SemaphoreType.DMA],
    )(_ks2_body)
    urows = ks2(src2d, u_nodes)

    es2d = pl.pallas_call(
        _ke2_body,
        grid=(E_PAD // CE,),
        in_specs=[pl.BlockSpec((CE, 1), lambda i: (i, 0)),
                  pl.BlockSpec((CE, 1), lambda i: (i, 0)),
                  pl.BlockSpec((CE, 1), lambda i: (i, 0)),
                  pl.BlockSpec((CE, 1), lambda i: (i, 0)),
                  pl.BlockSpec((CE, F), lambda i: (i, 0))],
        out_specs=pl.BlockSpec((CE // 128, 128), lambda i: (i, 0)),
        out_shape=jax.ShapeDtypeStruct((NROWS, 128), f32),
    )(evx, evy, evz, el_col, urows)

    ks3 = functools.partial(
        pl.kernel,
        out_type=jax.ShapeDtypeStruct((NW * N,), f32),
        mesh=_MESH,
        compiler_params=_SC_PARAMS,
        scratch_types=[pltpu.VMEM((N,), f32),
                       pltpu.VMEM((8, 128), f32),
                       pltpu.VMEM((8, 128), jnp.int32)],
    )(_ks3_body)
    d1p = ks3(dst2d, es2d)
    d1 = jnp.pad(jnp.sum(d1p.reshape(NW, N), axis=0), (0, N_PAD - N))
    e1_col = e1s2.reshape(N_PAD, 1) + d1[:, None]

    acc = pl.pallas_call(
        _kn3_body,
        grid=(N_PAD // CN,),
        in_specs=[pl.BlockSpec((CN, 1), lambda i: (i, 0)),
                  pl.BlockSpec((CN, 1), lambda i: (i, 0)),
                  pl.BlockSpec((CN, 1), lambda i: (i, 0)),
                  pl.BlockSpec((CN, 1), lambda i: (i, 0))],
        out_specs=pl.BlockSpec((G, 3), lambda i: (0, 0)),
        out_shape=jax.ShapeDtypeStruct((G, 3), f32),
    )(eat_col, e02.reshape(N_PAD, 1), e1_col, batch_pad)

    stacked = acc.T
    total = jnp.sum(stacked, axis=0)
    return (total, stacked)


# speedup vs baseline: 3.2018x; 3.2018x over previous
import functools

import jax
import jax.numpy as jnp
from jax import lax
from jax.experimental import pallas as pl
from jax.experimental.pallas import tpu as pltpu
from jax.experimental.pallas import tpu_sc as plsc

N = 50000
E = 800000
S = 10
C = 64
F = 32
G = 500
R_MAX = 5.0

CE = 2048
RB = CE // 128
CN = 1024
E_PAD = 819200
NROWS = E_PAD // 128
N_PAD = 51200
NNR = N_PAD // 128

NC, NS = 2, 16
NW = NC * NS
ROWS_PW = NROWS // NW

_MESH = plsc.VectorSubcoreMesh(core_axis_name="c", subcore_axis_name="s",
                               num_cores=NC, num_subcores=NS)
_SC_PARAMS = pltpu.CompilerParams(needs_layout_passes=False,
                                  use_tc_tiling_on_sc=False)


def _kn0_body(na_ref, ae_ref, sp_ref, eat_ref):
    na = na_ref[...]
    iota = lax.broadcasted_iota(jnp.int32, (S, 1), 0).astype(jnp.float32)
    spf = jnp.dot(na, iota)
    sp_ref[...] = (spf + 0.5).astype(jnp.int32).reshape(CN // 128, 128)
    eat_ref[...] = jnp.dot(na, ae_ref[...]).reshape(CN // 128, 128)


def _edge_feats3(evx_ref, evy_ref, evz_ref, el_ref):
    r = el_ref[...][:, None, :]
    rs = jnp.maximum(r, 1e-6)
    x = r * (1.0 / R_MAX)
    x2 = x * x
    x6 = x2 * x2 * x2
    env = 1.0 - 28.0 * x6 + 48.0 * x6 * x - 21.0 * x6 * x2
    env = jnp.where(x < 1.0, env, 0.0)
    ki = lax.broadcasted_iota(jnp.int32, (1, F, 1), 1)
    nrow = (ki // 4 + 1).astype(jnp.float32)
    lrow = ki % 4
    arg = (jnp.pi / R_MAX) * (rs * nrow)
    rb = (jnp.sqrt(2.0 / R_MAX) * jnp.sin(arg)) / rs
    ux = evx_ref[...][:, None, :] / rs
    uy = evy_ref[...][:, None, :] / rs
    uz = evz_ref[...][:, None, :] / rs
    sh = jnp.where(lrow == 0, 1.0,
                   jnp.where(lrow == 1, ux, jnp.where(lrow == 2, uy, uz)))
    return rb * env * sh


def _ke1_body(evx_ref, evy_ref, evz_ref, el_ref, sp_ref, wr0_ref, wn_ref,
              m3_ref):
    ef3 = _edge_feats3(evx_ref, evy_ref, evz_ref, el_ref)
    bdims = (((1,), (1,)), ((0,), (0,)))
    wr0b = jnp.broadcast_to(wr0_ref[...][None], (RB, F, C))
    rw3 = lax.dot_general(wr0b, ef3, bdims)
    sp3 = sp_ref[...][:, None, :]
    ai = lax.broadcasted_iota(jnp.int32, (1, S, 1), 1)
    oh3 = (sp3 == ai).astype(jnp.float32)
    wnb = jnp.broadcast_to(wn_ref[...][None], (RB, S, C))
    wn3 = lax.dot_general(wnb, oh3, bdims)
    m3_ref[...] = jnp.transpose(rw3 * wn3, (1, 0, 2))


def _kn2_body(agt_ref, sp_ref, wo0_ref, sk0_ref, wu_ref, sr1_ref,
              wr0c_ref, u_ref, e0_ref, e1s_ref):
    onehot = (sp_ref[...] == lax.broadcasted_iota(jnp.int32, (CN, S), 1)
              ).astype(jnp.float32)
    nf1 = (lax.dot_general(agt_ref[...], wo0_ref[...], (((0,), (0,)), ((), ())))
           + jnp.dot(onehot, sk0_ref[...]))
    e0_ref[...] = jnp.dot(nf1, wr0c_ref[...]).reshape(CN // 128, 128)
    u_ref[...] = lax.dot_general(wu_ref[...], nf1, (((0,), (1,)), ((), ())))
    e1s = jnp.sum(nf1 * jnp.dot(onehot, sr1_ref[...]), axis=1, keepdims=True)
    e1s_ref[...] = e1s.reshape(CN // 128, 128)


def _ke2_body(evx_ref, evy_ref, evz_ref, el_ref, ur_ref, es_ref):
    ef3 = _edge_feats3(evx_ref, evy_ref, evz_ref, el_ref)
    ur3 = jnp.transpose(ur_ref[...], (1, 0, 2))
    es_ref[...] = jnp.sum(ef3 * ur3, axis=1)


def _kn3_body(eat_ref, e0_ref, e1_ref, batch_ref, acc_ref):
    i = pl.program_id(0)
    onehot = (batch_ref[...] == lax.broadcasted_iota(jnp.int32, (CN, G), 1)
              ).astype(jnp.float32)
    cdims = (((0,), (0,)), ((), ()))
    a0 = lax.dot_general(onehot, eat_ref[...], cdims)
    a1 = lax.dot_general(onehot, e0_ref[...], cdims)
    a2 = lax.dot_general(onehot, e1_ref[...], cdims)
    col = lax.broadcasted_iota(jnp.int32, (1, 3), 1)
    sel0 = (col == 0).astype(jnp.float32)
    sel1 = (col == 1).astype(jnp.float32)
    sel2 = (col == 2).astype(jnp.float32)

    @pl.when(i == 0)
    def _():
        acc_ref[...] = jnp.zeros((G, 3), jnp.float32)

    acc_ref[...] += a0 * sel0 + a1 * sel1 + a2 * sel2


def _ks0_body(src2d, species_hbm, out_hbm, tab_v, idx_v, out_v):
    c = lax.axis_index("c")
    s = lax.axis_index("s")
    wid = s * NC + c
    rbase = wid * ROWS_PW
    pltpu.sync_copy(species_hbm, tab_v)

    def outer(jo, carry):
        pltpu.sync_copy(src2d.at[pl.ds(rbase + jo * 8, 8)], idx_v)
        for r in range(8):
            for g in range(8):
                idx = idx_v[r, pl.ds(g * 16, 16)]
                out_v[r, pl.ds(g * 16, 16)] = plsc.load_gather(tab_v, [idx])
        pltpu.sync_copy(out_v, out_hbm.at[pl.ds(rbase + jo * 8, 8)])
        return carry

    lax.fori_loop(0, ROWS_PW // 8, outer, 0)


_KS1_CH = 32

def _ks1_body(m3_hbm, dst2d, out_hbm, acc0, acc1, idx_v, v0, v1):
    c = lax.axis_index("c")
    s = lax.axis_index("s")
    wid = s * NC + c

    def z(i, carry):
        acc0[pl.ds(i * 16, 16)] = jnp.zeros((16,), jnp.float32)
        acc1[pl.ds(i * 16, 16)] = jnp.zeros((16,), jnp.float32)
        return carry

    lax.fori_loop(0, N_PAD // 16, z, 0)

    def outer(jo, carry):
        row0 = jo * _KS1_CH
        pltpu.sync_copy(dst2d.at[pl.ds(row0, _KS1_CH)], idx_v)
        pltpu.sync_copy(m3_hbm.at[2 * wid].at[pl.ds(row0, _KS1_CH)], v0)
        pltpu.sync_copy(m3_hbm.at[2 * wid + 1].at[pl.ds(row0, _KS1_CH)], v1)

        def row(r, carry2):
            for g in range(8):
                vidx = idx_v[r, pl.ds(g * 16, 16)]
                plsc.addupdate_scatter(acc0, [vidx], v0[r, pl.ds(g * 16, 16)])
                plsc.addupdate_scatter(acc1, [vidx], v1[r, pl.ds(g * 16, 16)])
            return carry2

        lax.fori_loop(0, _KS1_CH, row, 0)
        return carry

    lax.fori_loop(0, NROWS // _KS1_CH, outer, 0)
    pltpu.sync_copy(acc0, out_hbm.at[2 * wid])
    pltpu.sync_copy(acc1, out_hbm.at[2 * wid + 1])


def _ks2_body(src2d, ut_hbm, out_hbm, urow_v, idx_v, out_v):
    c = lax.axis_index("c")
    s = lax.axis_index("s")
    wid = s * NC + c
    pltpu.sync_copy(ut_hbm.at[wid], urow_v)

    def outer(jo, carry):
        row0 = jo * 32
        pltpu.sync_copy(src2d.at[pl.ds(row0, 32)], idx_v)

        def row(r, carry2):
            for g in range(8):
                idx = idx_v[r, pl.ds(g * 16, 16)]
                out_v[r, pl.ds(g * 16, 16)] = plsc.load_gather(urow_v, [idx])
            return carry2

        lax.fori_loop(0, 32, row, 0)
        pltpu.sync_copy(out_v, out_hbm.at[wid].at[pl.ds(row0, 32)])
        return carry

    lax.fori_loop(0, NROWS // 32, outer, 0)


def _ks3_body(dst2d, es2d, out_hbm, d1_v, val_v, idx_v):
    c = lax.axis_index("c")
    s = lax.axis_index("s")
    wid = s * NC + c
    rbase = wid * ROWS_PW

    def z(i, carry):
        d1_v[pl.ds(i * 16, 16)] = jnp.zeros((16,), jnp.float32)
        return carry

    lax.fori_loop(0, N // 16, z, 0)

    def outer(jo, carry):
        row0 = rbase + jo * 8
        pltpu.sync_copy(dst2d.at[pl.ds(row0, 8)], idx_v)
        pltpu.sync_copy(es2d.at[pl.ds(row0, 8)], val_v)
        for k in range(8):
            for g in range(8):
                vidx = idx_v[k, pl.ds(g * 16, 16)]
                vval = val_v[k, pl.ds(g * 16, 16)]
                plsc.addupdate_scatter(d1_v, [vidx], vval)
        return carry

    lax.fori_loop(0, ROWS_PW // 8, outer, 0)
    pltpu.sync_copy(d1_v, out_hbm.at[pl.ds(wid * N, N)])


def kernel(node_attrs, edge_vectors, edge_lengths, edge_index, batch,
           num_graphs, W_node, atomic_energies,
           W_r0, W_skip0, W_out0, w_read0,
           W_r1, W_skip1, W_out1, w_read1):
    f32 = jnp.float32
    skip0_tab = jnp.einsum('ac,acd->ad', W_node, W_skip0)
    v1 = W_out1 @ w_read1
    sr1_tab = jnp.einsum('acd,d->ac', W_skip1, w_read1)
    Wu = (W_r1 * v1[None, :]).T

    src = edge_index[0]
    dst = edge_index[1]
    pad = E_PAD - E
    src2d = jnp.pad(src, (0, pad)).astype(jnp.int32).reshape(NROWS, 128)
    dst2d = jnp.pad(dst, (0, pad)).astype(jnp.int32).reshape(NROWS, 128)
    el2 = jnp.pad(edge_lengths, (0, pad),
                  constant_values=2.0 * R_MAX).reshape(NROWS, 128)
    evx2 = jnp.pad(edge_vectors[:, 0], (0, pad)).reshape(NROWS, 128)
    evy2 = jnp.pad(edge_vectors[:, 1], (0, pad)).reshape(NROWS, 128)
    evz2 = jnp.pad(edge_vectors[:, 2], (0, pad)).reshape(NROWS, 128)
    na_pad = jnp.pad(node_attrs, ((0, N_PAD - N), (0, 0)))
    batch_pad = jnp.pad(batch.astype(jnp.int32), (0, N_PAD - N),
                        constant_values=G)[:, None]

    species2, eat2 = pl.pallas_call(
        _kn0_body,
        grid=(N_PAD // CN,),
        in_specs=[pl.BlockSpec((CN, S), lambda i: (i, 0)),
                  pl.BlockSpec((S, 1), lambda i: (0, 0))],
        out_specs=[pl.BlockSpec((CN // 128, 128), lambda i: (i, 0)),
                   pl.BlockSpec((CN // 128, 128), lambda i: (i, 0))],
        out_shape=[jax.ShapeDtypeStruct((NNR, 128), jnp.int32),
                   jax.ShapeDtypeStruct((NNR, 128), f32)],
    )(na_pad, atomic_energies[:, None])
    species_flat = species2.reshape(N_PAD)
    species_col = species2.reshape(N_PAD, 1)
    eat_col = eat2.reshape(N_PAD, 1)

    ks0 = functools.partial(
        pl.kernel,
        out_type=jax.ShapeDtypeStruct((NROWS, 128), jnp.int32),
        mesh=_MESH,
        compiler_params=_SC_PARAMS,
        scratch_types=[pltpu.VMEM((N_PAD,), jnp.int32),
                       pltpu.VMEM((8, 128), jnp.int32),
                       pltpu.VMEM((8, 128), jnp.int32)],
    )(_ks0_body)
    sp2d = ks0(src2d, species_flat)

    m3 = pl.pallas_call(
        _ke1_body,
        grid=(E_PAD // CE,),
        in_specs=[pl.BlockSpec((RB, 128), lambda i: (i, 0)),
                  pl.BlockSpec((RB, 128), lambda i: (i, 0)),
                  pl.BlockSpec((RB, 128), lambda i: (i, 0)),
                  pl.BlockSpec((RB, 128), lambda i: (i, 0)),
                  pl.BlockSpec((RB, 128), lambda i: (i, 0)),
                  pl.BlockSpec((F, C), lambda i: (0, 0)),
                  pl.BlockSpec((S, C), lambda i: (0, 0))],
        out_specs=pl.BlockSpec((C, RB, 128), lambda i: (0, i, 0)),
        out_shape=jax.ShapeDtypeStruct((C, NROWS, 128), f32),
    )(evx2, evy2, evz2, el2, sp2d, W_r0, W_node)

    ks1 = functools.partial(
        pl.kernel,
        out_type=jax.ShapeDtypeStruct((C, N_PAD), f32),
        mesh=_MESH,
        compiler_params=_SC_PARAMS,
        scratch_types=[pltpu.VMEM((N_PAD,), f32),
                       pltpu.VMEM((N_PAD,), f32),
                       pltpu.VMEM((_KS1_CH, 128), jnp.int32),
                       pltpu.VMEM((_KS1_CH, 128), f32),
                       pltpu.VMEM((_KS1_CH, 128), f32)],
    )(_ks1_body)
    agg_t = ks1(m3, dst2d)

    u_t, e02, e1s2 = pl.pallas_call(
        _kn2_body,
        grid=(N_PAD // CN,),
        in_specs=[pl.BlockSpec((C, CN), lambda i: (0, i)),
                  pl.BlockSpec((CN, 1), lambda i: (i, 0)),
                  pl.BlockSpec((C, C), lambda i: (0, 0)),
                  pl.BlockSpec((S, C), lambda i: (0, 0)),
                  pl.BlockSpec((C, F), lambda i: (0, 0)),
                  pl.BlockSpec((S, C), lambda i: (0, 0)),
                  pl.BlockSpec((C, 1), lambda i: (0, 0))],
        out_specs=[pl.BlockSpec((F, CN), lambda i: (0, i)),
                   pl.BlockSpec((CN // 128, 128), lambda i: (i, 0)),
                   pl.BlockSpec((CN // 128, 128), lambda i: (i, 0))],
        out_shape=[jax.ShapeDtypeStruct((F, N_PAD), f32),
                   jax.ShapeDtypeStruct((NNR, 128), f32),
                   jax.ShapeDtypeStruct((NNR, 128), f32)],
    )(agg_t, species_col, W_out0, skip0_tab, Wu, sr1_tab, w_read0[:, None])

    ks2 = functools.partial(
        pl.kernel,
        out_type=jax.ShapeDtypeStruct((F, NROWS, 128), f32),
        mesh=_MESH,
        compiler_params=_SC_PARAMS,
        scratch_types=[pltpu.VMEM((N_PAD,), f32),
                       pltpu.VMEM((32, 128), jnp.int32),
                       pltpu.VMEM((32, 128), f32)],
    )(_ks2_body)
    ur3 = ks2(src2d, u_t)

    es2d = pl.pallas_call(
        _ke2_body,
        grid=(E_PAD // CE,),
        in_specs=[pl.BlockSpec((RB, 128), lambda i: (i, 0)),
                  pl.BlockSpec((RB, 128), lambda i: (i, 0)),
                  pl.BlockSpec((RB, 128), lambda i: (i, 0)),
                  pl.BlockSpec((RB, 128), lambda i: (i, 0)),
                  pl.BlockSpec((F, RB, 128), lambda i: (0, i, 0))],
        out_specs=pl.BlockSpec((RB, 128), lambda i: (i, 0)),
        out_shape=jax.ShapeDtypeStruct((NROWS, 128), f32),
    )(evx2, evy2, evz2, el2, ur3)

    ks3 = functools.partial(
        pl.kernel,
        out_type=jax.ShapeDtypeStruct((NW * N,), f32),
        mesh=_MESH,
        compiler_params=_SC_PARAMS,
        scratch_types=[pltpu.VMEM((N,), f32),
                       pltpu.VMEM((8, 128), f32),
                       pltpu.VMEM((8, 128), jnp.int32)],
    )(_ks3_body)
    d1p = ks3(dst2d, es2d)
    d1 = jnp.pad(jnp.sum(d1p.reshape(NW, N), axis=0), (0, N_PAD - N))
    e1_col = e1s2.reshape(N_PAD, 1) + d1[:, None]

    acc = pl.pallas_call(
        _kn3_body,
        grid=(N_PAD // CN,),
        in_specs=[pl.BlockSpec((CN, 1), lambda i: (i, 0)),
                  pl.BlockSpec((CN, 1), lambda i: (i, 0)),
                  pl.BlockSpec((CN, 1), lambda i: (i, 0)),
                  pl.BlockSpec((CN, 1), lambda i: (i, 0))],
        out_specs=pl.BlockSpec((G, 3), lambda i: (0, 0)),
        out_shape=jax.ShapeDtypeStruct((G, 3), f32),
    )(eat_col, e02.reshape(N_PAD, 1), e1_col, batch_pad)

    stacked = acc.T
    total = jnp.sum(stacked, axis=0)
    return (total, stacked)
